# baseline (speedup 1.0000x reference)
import functools

import jax
import jax.numpy as jnp
from jax.experimental import pallas as pl
from jax.experimental.pallas import tpu as pltpu

D = 768
H = 2048
E = 8
S = 2048
TBLK = 256
NT = S // TBLK


def _moe_dense_kernel(x_ref, wg_ref, bg_ref, w1_ref, b1_ref, w2_ref, b2_ref,
                      gamma_ref, beta_ref, out_ref):
    e = pl.program_id(0)
    t = pl.program_id(1)
    xb = x_ref[...]

    h = jnp.dot(xb, w1_ref[0], preferred_element_type=jnp.float32)
    h = jnp.maximum(h + b1_ref[0], 0.0)
    o = jnp.dot(h, w2_ref[0], preferred_element_type=jnp.float32)
    o = o + b2_ref[0]

    logits = jnp.dot(xb, wg_ref[...], preferred_element_type=jnp.float32)
    logits = logits + bg_ref[...]
    m = jnp.max(logits, axis=-1, keepdims=True)
    ex = jnp.exp(logits - m)
    rw = ex / jnp.sum(ex, axis=-1, keepdims=True)
    ii = jax.lax.broadcasted_iota(jnp.int32, rw.shape, 1)
    m1 = jnp.max(rw, axis=-1, keepdims=True)
    e1 = jnp.min(jnp.where(rw == m1, ii, E), axis=-1, keepdims=True)
    rw2 = jnp.where(ii == e1, -1.0, rw)
    m2 = jnp.max(rw2, axis=-1, keepdims=True)
    e2 = jnp.min(jnp.where(rw2 == m2, ii, E), axis=-1, keepdims=True)
    w = jnp.where(ii == e1, m1, 0.0) + jnp.where(ii == e2, m2, 0.0)
    we = jnp.sum(jnp.where(ii == e, w, 0.0), axis=-1, keepdims=True)

    contrib = we * o
    row = pl.ds(t * TBLK, TBLK)

    @pl.when(e == 0)
    def _():
        out_ref[row, :] = contrib

    @pl.when(e > 0)
    def _():
        out_ref[row, :] = out_ref[row, :] + contrib

    @pl.when(e == E - 1)
    def _():
        out = out_ref[row, :] + xb
        mean = jnp.mean(out, axis=-1, keepdims=True)
        c = out - mean
        var = jnp.mean(c * c, axis=-1, keepdims=True)
        out_ref[row, :] = (c * jax.lax.rsqrt(var + 1e-5) * gamma_ref[...]
                           + beta_ref[...])


@jax.jit
def _moe_dense(x2, Wg, bg, W1, b1, W2, b2, gamma, beta):
    grid = (E, NT)
    return pl.pallas_call(
        _moe_dense_kernel,
        grid=grid,
        in_specs=[
            pl.BlockSpec((TBLK, D), lambda e, t: (t, 0)),
            pl.BlockSpec((D, E), lambda e, t: (0, 0)),
            pl.BlockSpec((1, E), lambda e, t: (0, 0)),
            pl.BlockSpec((1, D, H), lambda e, t: (e, 0, 0)),
            pl.BlockSpec((1, 1, H), lambda e, t: (e, 0, 0)),
            pl.BlockSpec((1, H, D), lambda e, t: (e, 0, 0)),
            pl.BlockSpec((1, 1, D), lambda e, t: (e, 0, 0)),
            pl.BlockSpec((1, D), lambda e, t: (0, 0)),
            pl.BlockSpec((1, D), lambda e, t: (0, 0)),
        ],
        out_specs=pl.BlockSpec((S, D), lambda e, t: (0, 0)),
        out_shape=jax.ShapeDtypeStruct((S, D), jnp.float32),
        compiler_params=pltpu.CompilerParams(
            dimension_semantics=("arbitrary", "arbitrary")),
    )(x2, Wg, bg, W1, b1, W2, b2, gamma, beta)


def kernel(x, Wg, bg, W1, b1, W2, b2, gamma, beta):
    x2 = x.reshape(S, D)
    out = _moe_dense(x2, Wg, bg.reshape(1, E), W1, b1.reshape(E, 1, H),
                     W2, b2.reshape(E, 1, D), gamma.reshape(1, D),
                     beta.reshape(1, D))
    return out.reshape(x.shape)
